# baseline (speedup 1.0000x reference)
import functools

import jax
import jax.numpy as jnp
from jax import lax
from jax.experimental import pallas as pl
from jax.experimental.pallas import tpu as pltpu
from jax.experimental.pallas import tpu_sc as plsc

NUM_CORES = 2
NUM_SUBCORES = 16
NUM_WORKERS = NUM_CORES * NUM_SUBCORES
BATCH = 16384
EMBED_DIM = 64
NUM_ENT = 1000000
BPW = BATCH // NUM_WORKERS
CHUNK = 128
NCHUNK = BPW // CHUNK
IDX_ROWS_PER_W = BPW // CHUNK

TC_W = 896
FULL_CHUNKS = NUM_ENT // TC_W
TAIL_W = NUM_ENT - FULL_CHUNKS * TC_W
CHUNK_ITERS = -(-FULL_CHUNKS // NUM_WORKERS)


def _transpose_chunk(src, dst, width):
    lanes64 = lax.iota(jnp.int32, 16) * EMBED_DIM

    def body(d, carry):
        for eb in range(width // 16):
            v = src[d, pl.ds(eb * 16, 16)]
            idx = lanes64 + (eb * 16 * EMBED_DIM + d)
            plsc.store_scatter(dst, [idx], v)
        return carry

    lax.fori_loop(0, EMBED_DIM, body, 0)


def _repack_body(et_h, tail_h, out_h, buf, outb, tail_in, tail_out,
                 isem, osem):
    wid = lax.axis_index("s") * NUM_CORES + lax.axis_index("c")

    def start_in(j):
        c = wid + NUM_WORKERS * j
        pltpu.async_copy(et_h.at[:, pl.ds(c * TC_W, TC_W)], buf, isem)

    start_in(0)

    def body(j, carry):
        c = wid + NUM_WORKERS * j

        @pl.when(c < FULL_CHUNKS)
        def _process():
            pltpu.make_async_copy(
                et_h.at[:, pl.ds(0, TC_W)], buf, isem).wait()

            @pl.when(j > 0)
            def _wait_out():
                pltpu.make_async_copy(
                    outb, out_h.at[pl.ds(0, TC_W * EMBED_DIM)], osem).wait()

            _transpose_chunk(buf, outb, TC_W)
            pltpu.async_copy(
                outb, out_h.at[pl.ds(c * TC_W * EMBED_DIM,
                                     TC_W * EMBED_DIM)], osem)

            @pl.when(c + NUM_WORKERS < FULL_CHUNKS)
            def _next():
                start_in(j + 1)
        return carry

    lax.fori_loop(0, CHUNK_ITERS, body, 0)

    pltpu.make_async_copy(
        outb, out_h.at[pl.ds(0, TC_W * EMBED_DIM)], osem).wait()

    @pl.when(wid == 4)
    def _tail():
        pltpu.sync_copy(tail_h, tail_in)
        _transpose_chunk(tail_in, tail_out, TAIL_W)
        pltpu.sync_copy(
            tail_out,
            out_h.at[pl.ds(FULL_CHUNKS * TC_W * EMBED_DIM,
                           TAIL_W * EMBED_DIM)])


def _score_body(head_h, rel_h, tail_h, ent_h, relemb_h, out_h,
                hidx, ridx, tidx, hrows, rrows, trows, outv, sem):
    wid = lax.axis_index("s") * NUM_CORES + lax.axis_index("c")
    rbase = wid * IDX_ROWS_PER_W

    pltpu.sync_copy(head_h.at[pl.ds(rbase, IDX_ROWS_PER_W)], hidx)
    pltpu.sync_copy(rel_h.at[pl.ds(rbase, IDX_ROWS_PER_W)], ridx)
    pltpu.sync_copy(tail_h.at[pl.ds(rbase, IDX_ROWS_PER_W)], tidx)

    cps = []
    for j in range(NCHUNK):
        cps.append(pltpu.async_copy(
            ent_h.at[hidx.at[j]], hrows.at[pl.ds(j * CHUNK, CHUNK)], sem))
        cps.append(pltpu.async_copy(
            relemb_h.at[ridx.at[j]], rrows.at[pl.ds(j * CHUNK, CHUNK)], sem))
        cps.append(pltpu.async_copy(
            ent_h.at[tidx.at[j]], trows.at[pl.ds(j * CHUNK, CHUNK)], sem))
    for cp in cps:
        cp.wait()

    lanes = lax.iota(jnp.int32, 16)
    dnums = lax.GatherDimensionNumbers(
        offset_dims=(), collapsed_slice_dims=(0,), start_index_map=(0,))

    def lane_sum(v):
        for s in (8, 4, 2, 1):
            perm = lax.gather(
                v, (lanes ^ s)[:, None], dimension_numbers=dnums,
                slice_sizes=(1,),
                mode=lax.GatherScatterMode.PROMISE_IN_BOUNDS)
            v = v + perm
        return v

    def group(g, carry):
        base = g * 16
        scores = jnp.zeros((16,), jnp.float32)
        for j in range(16):
            b = base + j
            acc = (hrows[b, pl.ds(0, 16)] * rrows[b, pl.ds(0, 16)]
                   * trows[b, pl.ds(0, 16)])
            for c in range(1, EMBED_DIM // 16):
                acc = acc + (hrows[b, pl.ds(c * 16, 16)]
                             * rrows[b, pl.ds(c * 16, 16)]
                             * trows[b, pl.ds(c * 16, 16)])
            scores = jnp.where(lanes == j, lane_sum(acc), scores)
        outv[pl.ds(base, 16)] = scores
        return carry

    lax.fori_loop(0, BPW // 16, group, 0)

    pltpu.sync_copy(outv, out_h.at[pl.ds(wid * BPW, BPW)])


@jax.jit
def kernel(head, relation, tail, entity_embeddings, relation_embeddings):
    h = head.astype(jnp.int32).reshape(BATCH // CHUNK, CHUNK)
    r = relation.astype(jnp.int32).reshape(BATCH // CHUNK, CHUNK)
    t = tail.astype(jnp.int32).reshape(BATCH // CHUNK, CHUNK)

    mesh = plsc.VectorSubcoreMesh(core_axis_name="c", subcore_axis_name="s")

    et = entity_embeddings.T
    tail_cols = et[:, FULL_CHUNKS * TC_W:]

    repack = functools.partial(
        pl.kernel,
        mesh=mesh,
        compiler_params=pltpu.CompilerParams(needs_layout_passes=False),
        out_type=jax.ShapeDtypeStruct((NUM_ENT * EMBED_DIM,), jnp.float32),
        scratch_types=[
            pltpu.VMEM((EMBED_DIM, TC_W), jnp.float32),
            pltpu.VMEM((TC_W * EMBED_DIM,), jnp.float32),
            pltpu.VMEM((EMBED_DIM, TAIL_W), jnp.float32),
            pltpu.VMEM((TAIL_W * EMBED_DIM,), jnp.float32),
            pltpu.SemaphoreType.DMA,
            pltpu.SemaphoreType.DMA,
        ],
    )(_repack_body)
    ent_lin = repack(et, tail_cols).reshape(NUM_ENT, EMBED_DIM)

    score = functools.partial(
        pl.kernel,
        mesh=mesh,
        compiler_params=pltpu.CompilerParams(use_tc_tiling_on_sc=False),
        out_type=jax.ShapeDtypeStruct((BATCH,), jnp.float32),
        scratch_types=[
            pltpu.VMEM((IDX_ROWS_PER_W, CHUNK), jnp.int32),
            pltpu.VMEM((IDX_ROWS_PER_W, CHUNK), jnp.int32),
            pltpu.VMEM((IDX_ROWS_PER_W, CHUNK), jnp.int32),
            pltpu.VMEM((BPW, EMBED_DIM), jnp.float32),
            pltpu.VMEM((BPW, EMBED_DIM), jnp.float32),
            pltpu.VMEM((BPW, EMBED_DIM), jnp.float32),
            pltpu.VMEM((BPW,), jnp.float32),
            pltpu.SemaphoreType.DMA,
        ],
    )(_score_body)
    return score(h, r, t, ent_lin, relation_embeddings)


# speedup vs baseline: 1.3430x; 1.3430x over previous
import functools

import jax
import jax.numpy as jnp
from jax import lax
from jax.experimental import pallas as pl
from jax.experimental.pallas import tpu as pltpu
from jax.experimental.pallas import tpu_sc as plsc

NUM_CORES = 2
NUM_SUBCORES = 16
NUM_WORKERS = NUM_CORES * NUM_SUBCORES
BATCH = 16384
EMBED_DIM = 64
NUM_ENT = 1000000
BPW = BATCH // NUM_WORKERS
CHUNK = 128
NCHUNK = BPW // CHUNK
IDX_ROWS_PER_W = BPW // CHUNK

TC_W = 896
FULL_CHUNKS = NUM_ENT // TC_W
TAIL_W = NUM_ENT - FULL_CHUNKS * TC_W
CHUNK_ITERS = -(-FULL_CHUNKS // NUM_WORKERS)


def _transpose_chunk(src, dst, width):
    lanes64 = lax.iota(jnp.int32, 16) * EMBED_DIM

    @plsc.parallel_loop(0, EMBED_DIM, unroll=4)
    def body(d):
        for eb in range(width // 16):
            v = src[d, pl.ds(eb * 16, 16)]
            idx = lanes64 + (eb * 16 * EMBED_DIM + d)
            plsc.store_scatter(dst, [idx], v)


def _repack_body(et_h, tail_h, out_h, buf, outb, tail_in, tail_out,
                 isem, osem):
    wid = lax.axis_index("s") * NUM_CORES + lax.axis_index("c")

    def start_in(j):
        c = wid + NUM_WORKERS * j
        pltpu.async_copy(et_h.at[:, pl.ds(c * TC_W, TC_W)], buf, isem)

    start_in(0)

    def body(j, carry):
        c = wid + NUM_WORKERS * j

        @pl.when(c < FULL_CHUNKS)
        def _process():
            pltpu.make_async_copy(
                et_h.at[:, pl.ds(0, TC_W)], buf, isem).wait()

            @pl.when(j > 0)
            def _wait_out():
                pltpu.make_async_copy(
                    outb, out_h.at[pl.ds(0, TC_W * EMBED_DIM)], osem).wait()

            _transpose_chunk(buf, outb, TC_W)
            pltpu.async_copy(
                outb, out_h.at[pl.ds(c * TC_W * EMBED_DIM,
                                     TC_W * EMBED_DIM)], osem)

            @pl.when(c + NUM_WORKERS < FULL_CHUNKS)
            def _next():
                start_in(j + 1)
        return carry

    lax.fori_loop(0, CHUNK_ITERS, body, 0)

    pltpu.make_async_copy(
        outb, out_h.at[pl.ds(0, TC_W * EMBED_DIM)], osem).wait()

    @pl.when(wid == 4)
    def _tail():
        pltpu.sync_copy(tail_h, tail_in)
        _transpose_chunk(tail_in, tail_out, TAIL_W)
        pltpu.sync_copy(
            tail_out,
            out_h.at[pl.ds(FULL_CHUNKS * TC_W * EMBED_DIM,
                           TAIL_W * EMBED_DIM)])


def _score_body(head_h, rel_h, tail_h, ent_h, relemb_h, out_h,
                hidx, ridx, tidx, hrows, rrows, trows, outv, sem):
    wid = lax.axis_index("s") * NUM_CORES + lax.axis_index("c")
    rbase = wid * IDX_ROWS_PER_W

    pltpu.sync_copy(head_h.at[pl.ds(rbase, IDX_ROWS_PER_W)], hidx)
    pltpu.sync_copy(rel_h.at[pl.ds(rbase, IDX_ROWS_PER_W)], ridx)
    pltpu.sync_copy(tail_h.at[pl.ds(rbase, IDX_ROWS_PER_W)], tidx)

    cps = []
    for j in range(NCHUNK):
        cps.append(pltpu.async_copy(
            ent_h.at[hidx.at[j]], hrows.at[pl.ds(j * CHUNK, CHUNK)], sem))
        cps.append(pltpu.async_copy(
            relemb_h.at[ridx.at[j]], rrows.at[pl.ds(j * CHUNK, CHUNK)], sem))
        cps.append(pltpu.async_copy(
            ent_h.at[tidx.at[j]], trows.at[pl.ds(j * CHUNK, CHUNK)], sem))
    for cp in cps:
        cp.wait()

    lanes = lax.iota(jnp.int32, 16)
    dnums = lax.GatherDimensionNumbers(
        offset_dims=(), collapsed_slice_dims=(0,), start_index_map=(0,))

    def lane_sum(v):
        for s in (8, 4, 2, 1):
            perm = lax.gather(
                v, (lanes ^ s)[:, None], dimension_numbers=dnums,
                slice_sizes=(1,),
                mode=lax.GatherScatterMode.PROMISE_IN_BOUNDS)
            v = v + perm
        return v

    def group(g, carry):
        base = g * 16
        scores = jnp.zeros((16,), jnp.float32)
        for j in range(16):
            b = base + j
            acc = (hrows[b, pl.ds(0, 16)] * rrows[b, pl.ds(0, 16)]
                   * trows[b, pl.ds(0, 16)])
            for c in range(1, EMBED_DIM // 16):
                acc = acc + (hrows[b, pl.ds(c * 16, 16)]
                             * rrows[b, pl.ds(c * 16, 16)]
                             * trows[b, pl.ds(c * 16, 16)])
            scores = jnp.where(lanes == j, lane_sum(acc), scores)
        outv[pl.ds(base, 16)] = scores
        return carry

    lax.fori_loop(0, BPW // 16, group, 0)

    pltpu.sync_copy(outv, out_h.at[pl.ds(wid * BPW, BPW)])


@jax.jit
def kernel(head, relation, tail, entity_embeddings, relation_embeddings):
    h = head.astype(jnp.int32).reshape(BATCH // CHUNK, CHUNK)
    r = relation.astype(jnp.int32).reshape(BATCH // CHUNK, CHUNK)
    t = tail.astype(jnp.int32).reshape(BATCH // CHUNK, CHUNK)

    mesh = plsc.VectorSubcoreMesh(core_axis_name="c", subcore_axis_name="s")

    et = entity_embeddings.T
    tail_cols = et[:, FULL_CHUNKS * TC_W:]

    repack = functools.partial(
        pl.kernel,
        mesh=mesh,
        compiler_params=pltpu.CompilerParams(needs_layout_passes=False),
        out_type=jax.ShapeDtypeStruct((NUM_ENT * EMBED_DIM,), jnp.float32),
        scratch_types=[
            pltpu.VMEM((EMBED_DIM, TC_W), jnp.float32),
            pltpu.VMEM((TC_W * EMBED_DIM,), jnp.float32),
            pltpu.VMEM((EMBED_DIM, TAIL_W), jnp.float32),
            pltpu.VMEM((TAIL_W * EMBED_DIM,), jnp.float32),
            pltpu.SemaphoreType.DMA,
            pltpu.SemaphoreType.DMA,
        ],
    )(_repack_body)
    ent_lin = repack(et, tail_cols).reshape(NUM_ENT, EMBED_DIM)

    score = functools.partial(
        pl.kernel,
        mesh=mesh,
        compiler_params=pltpu.CompilerParams(use_tc_tiling_on_sc=False),
        out_type=jax.ShapeDtypeStruct((BATCH,), jnp.float32),
        scratch_types=[
            pltpu.VMEM((IDX_ROWS_PER_W, CHUNK), jnp.int32),
            pltpu.VMEM((IDX_ROWS_PER_W, CHUNK), jnp.int32),
            pltpu.VMEM((IDX_ROWS_PER_W, CHUNK), jnp.int32),
            pltpu.VMEM((BPW, EMBED_DIM), jnp.float32),
            pltpu.VMEM((BPW, EMBED_DIM), jnp.float32),
            pltpu.VMEM((BPW, EMBED_DIM), jnp.float32),
            pltpu.VMEM((BPW,), jnp.float32),
            pltpu.SemaphoreType.DMA,
        ],
    )(_score_body)
    return score(h, r, t, ent_lin, relation_embeddings)


# speedup vs baseline: 2.2231x; 1.6553x over previous
import functools

import jax
import jax.numpy as jnp
from jax import lax
from jax.experimental import pallas as pl
from jax.experimental.pallas import tpu as pltpu
from jax.experimental.pallas import tpu_sc as plsc

NUM_CORES = 2
NUM_SUBCORES = 16
NUM_WORKERS = NUM_CORES * NUM_SUBCORES
BATCH = 16384
EMBED_DIM = 64
BPW = BATCH // NUM_WORKERS
CHUNK = 128
NCHUNK = BPW // CHUNK
IDX_ROWS_PER_W = BPW // CHUNK


def _sc_body(head_h, rel_h, tail_h, ent_h, relemb_h, out_h,
             hidx, ridx, tidx, hrows, rrows, trows, outv, sem):
    wid = lax.axis_index("s") * NUM_CORES + lax.axis_index("c")
    rbase = wid * IDX_ROWS_PER_W

    pltpu.sync_copy(head_h.at[pl.ds(rbase, IDX_ROWS_PER_W)], hidx)
    pltpu.sync_copy(rel_h.at[pl.ds(rbase, IDX_ROWS_PER_W)], ridx)
    pltpu.sync_copy(tail_h.at[pl.ds(rbase, IDX_ROWS_PER_W)], tidx)

    cps = []
    for j in range(NCHUNK):
        cps.append(pltpu.async_copy(
            ent_h.at[hidx.at[j]], hrows.at[pl.ds(j * CHUNK, CHUNK)], sem))
        cps.append(pltpu.async_copy(
            relemb_h.at[ridx.at[j]], rrows.at[pl.ds(j * CHUNK, CHUNK)], sem))
        cps.append(pltpu.async_copy(
            ent_h.at[tidx.at[j]], trows.at[pl.ds(j * CHUNK, CHUNK)], sem))
    for cp in cps:
        cp.wait()

    lanes = lax.iota(jnp.int32, 16)
    dnums = lax.GatherDimensionNumbers(
        offset_dims=(), collapsed_slice_dims=(0,), start_index_map=(0,))

    def lane_sum(v):
        for s in (8, 4, 2, 1):
            perm = lax.gather(
                v, (lanes ^ s)[:, None], dimension_numbers=dnums,
                slice_sizes=(1,),
                mode=lax.GatherScatterMode.PROMISE_IN_BOUNDS)
            v = v + perm
        return v

    def group(g, carry):
        base = g * 16
        scores = jnp.zeros((16,), jnp.float32)
        for j in range(16):
            b = base + j
            acc = (hrows[b, pl.ds(0, 16)] * rrows[b, pl.ds(0, 16)]
                   * trows[b, pl.ds(0, 16)])
            for c in range(1, EMBED_DIM // 16):
                acc = acc + (hrows[b, pl.ds(c * 16, 16)]
                             * rrows[b, pl.ds(c * 16, 16)]
                             * trows[b, pl.ds(c * 16, 16)])
            scores = jnp.where(lanes == j, lane_sum(acc), scores)
        outv[pl.ds(base, 16)] = scores
        return carry

    lax.fori_loop(0, BPW // 16, group, 0)

    pltpu.sync_copy(outv, out_h.at[pl.ds(wid * BPW, BPW)])


@jax.jit
def kernel(head, relation, tail, entity_embeddings, relation_embeddings):
    h = head.astype(jnp.int32).reshape(BATCH // CHUNK, CHUNK)
    r = relation.astype(jnp.int32).reshape(BATCH // CHUNK, CHUNK)
    t = tail.astype(jnp.int32).reshape(BATCH // CHUNK, CHUNK)

    mesh = plsc.VectorSubcoreMesh(core_axis_name="c", subcore_axis_name="s")
    run = functools.partial(
        pl.kernel,
        mesh=mesh,
        compiler_params=pltpu.CompilerParams(use_tc_tiling_on_sc=False),
        out_type=jax.ShapeDtypeStruct((BATCH,), jnp.float32),
        scratch_types=[
            pltpu.VMEM((IDX_ROWS_PER_W, CHUNK), jnp.int32),
            pltpu.VMEM((IDX_ROWS_PER_W, CHUNK), jnp.int32),
            pltpu.VMEM((IDX_ROWS_PER_W, CHUNK), jnp.int32),
            pltpu.VMEM((BPW, EMBED_DIM), jnp.float32),
            pltpu.VMEM((BPW, EMBED_DIM), jnp.float32),
            pltpu.VMEM((BPW, EMBED_DIM), jnp.float32),
            pltpu.VMEM((BPW,), jnp.float32),
            pltpu.SemaphoreType.DMA,
        ],
    )(_sc_body)
    return run(h, r, t, entity_embeddings, relation_embeddings)
